# baseline (speedup 1.0000x reference)
import jax
import jax.numpy as jnp
from jax import lax
from jax.experimental import pallas as pl
from jax.experimental.pallas import tpu as pltpu

_S = 2
_S3 = _S * _S * _S
_NBUF = 4
_NBUF2 = 8


def _data_body(data_ref, out_any, dbuf, sems):
    i = pl.program_id(0)
    nsteps = pl.num_programs(0)
    bn = data_ref.shape[0]
    c = data_ref.shape[1]
    slot = lax.rem(i, _NBUF)

    @pl.when(i >= _NBUF)
    def _wait_prev():
        pltpu.make_async_copy(
            dbuf.at[slot],
            out_any.at[pl.ds((i - _NBUF) * bn, bn)],
            sems.at[slot],
        ).wait()

    dbuf[slot] = jnp.broadcast_to(data_ref[...][:, None, :], (bn, _S3, c))
    pltpu.make_async_copy(
        dbuf.at[slot],
        out_any.at[pl.ds(i * bn, bn)],
        sems.at[slot],
    ).start()

    @pl.when(i == nsteps - 1)
    def _drain():
        for k in range(_NBUF):
            step = nsteps - _NBUF + k
            s = lax.rem(step, _NBUF)
            pltpu.make_async_copy(
                dbuf.at[s],
                out_any.at[pl.ds(step * bn, bn)],
                sems.at[s],
            ).wait()


def _ijk_body(ijk_any, joff_any, ijk_out, joff_out, src, ibuf, sems2, jbuf, sem1):
    i = pl.program_id(0)
    nsteps = pl.num_programs(0)
    bn = ibuf.shape[1]
    slot = lax.rem(i, _NBUF2)

    @pl.when(i == 0)
    def _stage():
        cp = pltpu.make_async_copy(ijk_any, src, sem1)
        cp.start()
        cp.wait()
        cpj = pltpu.make_async_copy(joff_any, jbuf, sem1)
        cpj.start()
        cpj.wait()
        jbuf[...] = jbuf[...] * _S3
        pltpu.make_async_copy(jbuf, joff_out, sem1).start()

    @pl.when(i >= _NBUF2)
    def _wait_prev():
        pltpu.make_async_copy(
            ibuf.at[slot],
            ijk_out.at[pl.ds((i - _NBUF2) * bn, bn)],
            sems2.at[slot],
        ).wait()

    kidx = lax.broadcasted_iota(jnp.int32, (bn, _S3, 3), 1)
    cidx = lax.broadcasted_iota(jnp.int32, (bn, _S3, 3), 2)
    off = lax.shift_right_logical(kidx, 2 - cidx) & 1
    ibuf[slot] = src[pl.ds(i * bn, bn), :][:, None, :] * _S + off
    pltpu.make_async_copy(
        ibuf.at[slot],
        ijk_out.at[pl.ds(i * bn, bn)],
        sems2.at[slot],
    ).start()

    @pl.when(i == nsteps - 1)
    def _drain():
        for k in range(_NBUF2):
            step = nsteps - _NBUF2 + k
            s = lax.rem(step, _NBUF2)
            pltpu.make_async_copy(
                ibuf.at[s],
                ijk_out.at[pl.ds(step * bn, bn)],
                sems2.at[s],
            ).wait()
        pltpu.make_async_copy(jbuf, joff_out, sem1).wait()


def kernel(coarse_data, coarse_ijk, joffsets):
    n, c = coarse_data.shape
    nj = joffsets.shape[0]
    bn = 1024
    grid = n // bn
    bn2 = 512
    grid2 = n // bn2

    fine3 = pl.pallas_call(
        _data_body,
        grid=(grid,),
        in_specs=[pl.BlockSpec((bn, c), lambda i: (i, 0))],
        out_specs=pl.BlockSpec(memory_space=pl.ANY),
        out_shape=jax.ShapeDtypeStruct((n, _S3, c), coarse_data.dtype),
        scratch_shapes=[
            pltpu.VMEM((_NBUF, bn, _S3, c), coarse_data.dtype),
            pltpu.SemaphoreType.DMA((_NBUF,)),
        ],
    )(coarse_data)

    ijk3, joff2 = pl.pallas_call(
        _ijk_body,
        grid=(grid2,),
        in_specs=[
            pl.BlockSpec(memory_space=pl.ANY),
            pl.BlockSpec(memory_space=pl.ANY),
        ],
        out_specs=[
            pl.BlockSpec(memory_space=pl.ANY),
            pl.BlockSpec(memory_space=pl.ANY),
        ],
        out_shape=[
            jax.ShapeDtypeStruct((n, _S3, 3), coarse_ijk.dtype),
            jax.ShapeDtypeStruct((1, nj), joffsets.dtype),
        ],
        scratch_shapes=[
            pltpu.VMEM((n, 3), jnp.int32),
            pltpu.VMEM((_NBUF2, bn2, _S3, 3), jnp.int32),
            pltpu.SemaphoreType.DMA((_NBUF2,)),
            pltpu.VMEM((1, nj), jnp.int32),
            pltpu.SemaphoreType.DMA,
        ],
    )(coarse_ijk, joffsets.reshape(1, nj))

    return (
        fine3.reshape(n * _S3, c),
        ijk3,
        joff2.reshape(nj),
    )


# speedup vs baseline: 1.0443x; 1.0443x over previous
import jax
import jax.numpy as jnp
from jax import lax
from jax.experimental import pallas as pl
from jax.experimental.pallas import tpu as pltpu

_S = 2
_S3 = _S * _S * _S
_NBUF = 4


def _body(data_ref, ijk_ref, joff_any, out_any, ijk_any, joff_out,
          dbuf, sems, ibuf, sems2, jbuf, sem1):
    i = pl.program_id(0)
    nsteps = pl.num_programs(0)
    bn = data_ref.shape[0]
    c = data_ref.shape[1]
    slot = lax.rem(i, _NBUF)

    @pl.when(i >= _NBUF)
    def _wait_prev():
        pltpu.make_async_copy(
            dbuf.at[slot],
            out_any.at[pl.ds((i - _NBUF) * bn, bn)],
            sems.at[slot],
        ).wait()
        pltpu.make_async_copy(
            ibuf.at[slot],
            ijk_any.at[pl.ds((i - _NBUF) * bn, bn)],
            sems2.at[slot],
        ).wait()

    dbuf[slot] = jnp.broadcast_to(data_ref[...][:, None, :], (bn, _S3, c))
    pltpu.make_async_copy(
        dbuf.at[slot],
        out_any.at[pl.ds(i * bn, bn)],
        sems.at[slot],
    ).start()

    kidx = lax.broadcasted_iota(jnp.int32, (bn, _S3, 3), 1)
    cidx = lax.broadcasted_iota(jnp.int32, (bn, _S3, 3), 2)
    off = lax.shift_right_logical(kidx, 2 - cidx) & 1
    ibuf[slot] = ijk_ref[...][:, None, :] * _S + off
    pltpu.make_async_copy(
        ibuf.at[slot],
        ijk_any.at[pl.ds(i * bn, bn)],
        sems2.at[slot],
    ).start()

    @pl.when(i == 0)
    def _joff_once():
        cp = pltpu.make_async_copy(joff_any, jbuf, sem1)
        cp.start()
        cp.wait()
        jbuf[...] = jbuf[...] * _S3
        pltpu.make_async_copy(jbuf, joff_out, sem1).start()

    @pl.when(i == nsteps - 1)
    def _drain():
        for k in range(_NBUF):
            step = nsteps - _NBUF + k
            s = lax.rem(step, _NBUF)
            pltpu.make_async_copy(
                dbuf.at[s],
                out_any.at[pl.ds(step * bn, bn)],
                sems.at[s],
            ).wait()
            pltpu.make_async_copy(
                ibuf.at[s],
                ijk_any.at[pl.ds(step * bn, bn)],
                sems2.at[s],
            ).wait()
        pltpu.make_async_copy(jbuf, joff_out, sem1).wait()


def kernel(coarse_data, coarse_ijk, joffsets):
    n, c = coarse_data.shape
    nj = joffsets.shape[0]
    bn = 1024
    grid = n // bn

    fine3, ijk3, joff2 = pl.pallas_call(
        _body,
        grid=(grid,),
        in_specs=[
            pl.BlockSpec((bn, c), lambda i: (i, 0)),
            pl.BlockSpec((bn, 3), lambda i: (i, 0)),
            pl.BlockSpec(memory_space=pl.ANY),
        ],
        out_specs=[
            pl.BlockSpec(memory_space=pl.ANY),
            pl.BlockSpec(memory_space=pl.ANY),
            pl.BlockSpec(memory_space=pl.ANY),
        ],
        out_shape=[
            jax.ShapeDtypeStruct((n, _S3, c), coarse_data.dtype),
            jax.ShapeDtypeStruct((n, _S3, 3), coarse_ijk.dtype),
            jax.ShapeDtypeStruct((1, nj), joffsets.dtype),
        ],
        scratch_shapes=[
            pltpu.VMEM((_NBUF, bn, _S3, c), coarse_data.dtype),
            pltpu.SemaphoreType.DMA((_NBUF,)),
            pltpu.VMEM((_NBUF, bn, _S3, 3), jnp.int32),
            pltpu.SemaphoreType.DMA((_NBUF,)),
            pltpu.VMEM((1, nj), jnp.int32),
            pltpu.SemaphoreType.DMA,
        ],
    )(coarse_data, coarse_ijk, joffsets.reshape(1, nj))
    return (
        fine3.reshape(n * _S3, c),
        ijk3.reshape(n * _S3, 3),
        joff2.reshape(nj),
    )
